# baseline (speedup 1.0000x reference)
import functools

import jax
import jax.numpy as jnp
from jax import lax
from jax.experimental import pallas as pl
from jax.experimental.pallas import tpu as pltpu
from jax.experimental.pallas import tpu_sc as plsc

_NUM_WORKERS = 32
_TSLAB = 40
_TSUB = 4


@functools.lru_cache(maxsize=None)
def _make_sc_gather(S, T, V, D):
    IPW = S // _NUM_WORKERS
    n_slabs = T // _TSLAB
    n_subs = _TSLAB // _TSUB
    n_pairs = n_subs // 2
    rows_per_sub = _TSUB * IPW
    vpt = IPW // 16
    mesh = plsc.VectorSubcoreMesh(core_axis_name="c", subcore_axis_name="s")

    @functools.partial(
        pl.kernel,
        mesh=mesh,
        out_type=jax.ShapeDtypeStruct((T, 2, S // 128, 8, 128), jnp.int32),
        scratch_types=[
            pltpu.VMEM((IPW, _TSLAB), jnp.int32),
            [pltpu.VMEM((rows_per_sub,), jnp.int32)] * 2,
            [pltpu.VMEM((rows_per_sub, D), jnp.bfloat16)] * 2,
            [pltpu.VMEM((2, 5, 8, 129), jnp.int32)] * 2,
            [pltpu.SemaphoreType.DMA] * 2,
            [pltpu.SemaphoreType.DMA] * 2,
        ],
        compiler_params=pltpu.CompilerParams(
            use_tc_tiling_on_sc=False, needs_layout_passes=False
        ),
    )
    def sc_kernel(ids_hbm, table_hbm, out_hbm, slab_v, gidxs, rowss,
                  stgs, gsems, wsems):
        wid = lax.axis_index("s") * 2 + lax.axis_index("c")
        i0 = wid * IPW
        hi0 = wid * (IPW // 128)
        iota = lax.iota(jnp.int32, 16)
        i_dpt = iota // 8
        i_dp8 = lax.rem(iota, 8)

        def build(g, sub_local):
            def bloop(k4, c):
                for u in range(4):
                    k = k4 * 4 + u
                    t_local = sub_local * _TSUB + k // vpt
                    i_base = lax.rem(k, vpt) * 16
                    vals = plsc.load_gather(
                        slab_v, [i_base + iota, iota * 0 + t_local])
                    gidxs[g][pl.ds(k * 16, 16)] = vals
                return c

            lax.fori_loop(0, rows_per_sub // 16 // 4, bloop, 0)

        def gather_start(g):
            pltpu.async_copy(table_hbm.at[gidxs[g]], rowss[g], gsems[g])

        def gather_wait(g):
            pltpu.make_async_copy(
                table_hbm.at[gidxs[g]], rowss[g], gsems[g]).wait()

        def drain(g, sub_global):
            for tt in range(_TSUB):
                b = tt % 2
                t_g = sub_global * _TSUB + tt
                if tt >= 2:
                    pltpu.make_async_copy(
                        stgs[b].at[:, pl.ds(0, 4), :, pl.ds(0, 128)],
                        out_hbm.at[t_g - 2, :, pl.ds(hi0, 4), :, :],
                        wsems[b],
                    ).wait()

                def xpose(r8, c):
                    hi = (r8 * 8) // 128
                    il_base = lax.rem(r8 * 8, 128)
                    for u in range(8):
                        r = r8 * 8 + u
                        v32 = rowss[g][tt * IPW + r, :]
                        w16 = plsc.bitcast(v32, jnp.int32)
                        plsc.store_scatter(
                            stgs[b],
                            [i_dpt, iota * 0 + hi, i_dp8,
                             iota * 0 + (il_base + u)],
                            w16,
                        )
                    return c

                lax.fori_loop(0, IPW // 8, xpose, 0)
                pltpu.async_copy(
                    stgs[b].at[:, pl.ds(0, 4), :, pl.ds(0, 128)],
                    out_hbm.at[t_g, :, pl.ds(hi0, 4), :, :],
                    wsems[b],
                )
            for tt in range(2, 4):
                b = tt % 2
                t_g = sub_global * _TSUB + tt
                pltpu.make_async_copy(
                    stgs[b].at[:, pl.ds(0, 4), :, pl.ds(0, 128)],
                    out_hbm.at[t_g, :, pl.ds(hi0, 4), :, :],
                    wsems[b],
                ).wait()

        def do_slab(slab_i, carry):
            pltpu.sync_copy(
                ids_hbm.at[pl.ds(i0, IPW), pl.ds(slab_i * _TSLAB, _TSLAB)],
                slab_v,
            )
            t_base = slab_i * n_subs

            def do_pair(sp, carry2):
                build(0, 2 * sp)
                gather_start(0)
                build(1, 2 * sp + 1)
                gather_start(1)
                gather_wait(0)
                drain(0, t_base + 2 * sp)
                gather_wait(1)
                drain(1, t_base + 2 * sp + 1)
                return carry2

            lax.fori_loop(0, n_pairs, do_pair, 0)
            return carry

        lax.fori_loop(0, n_slabs, do_slab, 0)

    return sc_kernel


@functools.lru_cache(maxsize=None)
def _make_tc_wtranspose(V, D):
    BLK = 2048

    def body(x_ref, z_ref):
        z_ref[...] = x_ref[...].T

    return pl.pallas_call(
        body,
        grid=(V // BLK,),
        in_specs=[pl.BlockSpec((D, BLK), lambda i: (0, i))],
        out_specs=pl.BlockSpec((BLK, D), lambda i: (i, 0)),
        out_shape=jax.ShapeDtypeStruct((V, D), jnp.bfloat16),
    )


@functools.lru_cache(maxsize=None)
def _make_tc_pairsplit(S, T, D):
    def body(x_ref, z_ref):
        x = x_ref[0]
        xu = lax.bitcast_convert_type(x, jnp.uint32)
        lo = (xu & jnp.uint32(0xFFFF)).astype(jnp.uint16)
        hi = (xu >> jnp.uint32(16)).astype(jnp.uint16)
        y = jnp.concatenate([lo[:, None, :], hi[:, None, :]], axis=1)
        z_ref[0] = lax.bitcast_convert_type(y.reshape(D, S // 4), jnp.bfloat16)

    return pl.pallas_call(
        body,
        grid=(T, 4),
        in_specs=[pl.BlockSpec((1, D // 2, S // 4), lambda i, j: (i, 0, j))],
        out_specs=pl.BlockSpec((1, D, S // 4), lambda i, j: (i, 0, j)),
        out_shape=jax.ShapeDtypeStruct((T, D, S), jnp.bfloat16),
    )


def kernel(ids, weight):
    S, T = ids.shape
    V, D = weight.shape
    ids = ids.astype(jnp.int32)
    o = _make_sc_gather(S, T, V, D)(ids, weight)
    oo = o.transpose(0, 1, 3, 2, 4).reshape(T, D // 2, S)
    z = _make_tc_pairsplit(S, T, D)(oo)
    return jnp.transpose(z, (2, 0, 1))


# speedup vs baseline: 1.1852x; 1.1852x over previous
import functools

import jax
import jax.numpy as jnp
from jax import lax
from jax.experimental import pallas as pl
from jax.experimental.pallas import tpu as pltpu
from jax.experimental.pallas import tpu_sc as plsc

_NUM_WORKERS = 32
_TSLAB = 40
_TSUB = 4


@functools.lru_cache(maxsize=None)
def _make_sc_gather(S, T, V, D):
    IPW = S // _NUM_WORKERS
    n_slabs = T // _TSLAB
    n_subs = _TSLAB // _TSUB
    n_pairs = n_subs // 2
    rows_per_sub = _TSUB * IPW
    vpt = IPW // 16
    mesh = plsc.VectorSubcoreMesh(core_axis_name="c", subcore_axis_name="s")

    @functools.partial(
        pl.kernel,
        mesh=mesh,
        out_type=jax.ShapeDtypeStruct((T, 2, S // 128, 8, 128), jnp.int32),
        scratch_types=[
            pltpu.VMEM((IPW, _TSLAB), jnp.int32),
            [pltpu.VMEM((rows_per_sub,), jnp.int32)] * 2,
            [pltpu.VMEM((rows_per_sub, D), jnp.bfloat16)] * 2,
            [pltpu.VMEM((2, 5, 8, 129), jnp.int32)] * 2,
            [pltpu.SemaphoreType.DMA] * 2,
            [pltpu.SemaphoreType.DMA] * 2,
        ],
        compiler_params=pltpu.CompilerParams(
            use_tc_tiling_on_sc=False, needs_layout_passes=False
        ),
    )
    def sc_kernel(ids_hbm, table_hbm, out_hbm, slab_v, gidxs, rowss,
                  stgs, gsems, wsems):
        wid = lax.axis_index("s") * 2 + lax.axis_index("c")
        i0 = wid * IPW
        hi0 = wid * (IPW // 128)
        iota = lax.iota(jnp.int32, 16)
        i_dpt = iota // 8
        i_dp8 = lax.rem(iota, 8)

        def build(g, sub_local):
            def bloop(k4, c):
                for u in range(4):
                    k = k4 * 4 + u
                    t_local = sub_local * _TSUB + k // vpt
                    i_base = lax.rem(k, vpt) * 16
                    vals = plsc.load_gather(
                        slab_v, [i_base + iota, iota * 0 + t_local])
                    gidxs[g][pl.ds(k * 16, 16)] = vals
                return c

            lax.fori_loop(0, rows_per_sub // 16 // 4, bloop, 0)

        def gather_start(g):
            pltpu.async_copy(table_hbm.at[gidxs[g]], rowss[g], gsems[g])

        def gather_wait(g):
            pltpu.make_async_copy(
                table_hbm.at[gidxs[g]], rowss[g], gsems[g]).wait()

        def drain(g, sub_global):
            for tt in range(_TSUB):
                b = tt % 2
                t_g = sub_global * _TSUB + tt
                if tt >= 2:
                    pltpu.make_async_copy(
                        stgs[b].at[:, pl.ds(0, 4), :, pl.ds(0, 128)],
                        out_hbm.at[t_g - 2, :, pl.ds(hi0, 4), :, :],
                        wsems[b],
                    ).wait()

                def xpose(r8, c):
                    hi = (r8 * 8) // 128
                    il_base = lax.rem(r8 * 8, 128)
                    for u in range(8):
                        r = r8 * 8 + u
                        v32 = rowss[g][tt * IPW + r, :]
                        w16 = plsc.bitcast(v32, jnp.int32)
                        plsc.store_scatter(
                            stgs[b],
                            [i_dpt, iota * 0 + hi, i_dp8,
                             iota * 0 + (il_base + u)],
                            w16,
                        )
                    return c

                lax.fori_loop(0, IPW // 8, xpose, 0)
                pltpu.async_copy(
                    stgs[b].at[:, pl.ds(0, 4), :, pl.ds(0, 128)],
                    out_hbm.at[t_g, :, pl.ds(hi0, 4), :, :],
                    wsems[b],
                )
            for tt in range(2, 4):
                b = tt % 2
                t_g = sub_global * _TSUB + tt
                pltpu.make_async_copy(
                    stgs[b].at[:, pl.ds(0, 4), :, pl.ds(0, 128)],
                    out_hbm.at[t_g, :, pl.ds(hi0, 4), :, :],
                    wsems[b],
                ).wait()

        def do_slab(slab_i, carry):
            pltpu.sync_copy(
                ids_hbm.at[pl.ds(i0, IPW), pl.ds(slab_i * _TSLAB, _TSLAB)],
                slab_v,
            )
            t_base = slab_i * n_subs

            def do_pair(sp, carry2):
                build(0, 2 * sp)
                gather_start(0)
                build(1, 2 * sp + 1)
                gather_start(1)
                gather_wait(0)
                drain(0, t_base + 2 * sp)
                gather_wait(1)
                drain(1, t_base + 2 * sp + 1)
                return carry2

            lax.fori_loop(0, n_pairs, do_pair, 0)
            return carry

        lax.fori_loop(0, n_slabs, do_slab, 0)

    return sc_kernel


@functools.lru_cache(maxsize=None)
def _make_tc_pairsplit(S, T, D):
    def body(x_ref, z_ref):
        x = x_ref[...]
        xu = lax.bitcast_convert_type(x, jnp.uint32)
        lo = (xu & jnp.uint32(0xFFFF)).astype(jnp.uint16)
        hi = (xu >> jnp.uint32(16)).astype(jnp.uint16)
        y = jnp.concatenate([lo[:, :, None, :], hi[:, :, None, :]], axis=2)
        z_ref[...] = lax.bitcast_convert_type(
            y.reshape(2, D, S), jnp.bfloat16)

    return pl.pallas_call(
        body,
        grid=(T // 2,),
        in_specs=[pl.BlockSpec((2, D // 2, S), lambda i: (i, 0, 0))],
        out_specs=pl.BlockSpec((2, D, S), lambda i: (i, 0, 0)),
        out_shape=jax.ShapeDtypeStruct((T, D, S), jnp.bfloat16),
    )


def kernel(ids, weight):
    S, T = ids.shape
    V, D = weight.shape
    ids = ids.astype(jnp.int32)
    o = _make_sc_gather(S, T, V, D)(ids, weight)
    oo = o.transpose(0, 1, 3, 2, 4).reshape(T, D // 2, S)
    z = _make_tc_pairsplit(S, T, D)(oo)
    return jnp.transpose(z, (2, 0, 1))
